# baseline (speedup 1.0000x reference)
import jax
import jax.numpy as jnp
from jax import lax
from jax.experimental import pallas as pl
from jax.experimental.pallas import tpu as pltpu
from jax.experimental.pallas import tpu_sc as plsc

N = 100000
D = 256
G = 512
DA = 16
L = 16
NK = D // L

NC = 2
NS = 16
NW = NC * NS

CHUNK = 128
NFCH = N // CHUNK
TAILROWS = N - NFCH * CHUNK
FULL_SLOTS = 24
LAST_W = NFCH + 1 - FULL_SLOTS * NW + NW - 1
ZR = 32


def _sc_body(x_hbm, ids_hbm, part_hbm, rows0_v, rows1_v, stage_v, zero_v,
             idv_v, sem0, sem1):
  cid = lax.axis_index("c")
  sid = lax.axis_index("s")
  wid = sid * NC + cid

  iota = lax.iota(jnp.int32, L)
  zvec = jnp.zeros((L,), jnp.float32)

  def chunk_of(s):
    return wid + NW * s

  def fetch(s, buf, sem):
    pltpu.async_copy(x_hbm.at[pl.ds(chunk_of(s) * CHUNK, CHUNK)], buf, sem)

  def wait(s, buf, sem):
    pltpu.make_async_copy(
        x_hbm.at[pl.ds(chunk_of(s) * CHUNK, CHUNK)], buf, sem).wait()

  fetch(0, rows0_v, sem0)

  @pl.loop(0, ZR)
  def _zr(r):
    @pl.loop(0, D, step=L)
    def _zc(j):
      zero_v[r, pl.ds(j, L)] = zvec

  for z in range(G // ZR):
    pltpu.sync_copy(zero_v, part_hbm.at[wid].at[pl.ds(z * ZR, ZR)])

  def flush(seg, acc):
    for k in range(NK):
      stage_v[0, pl.ds(k * L, L)] = acc[k]
    pltpu.sync_copy(stage_v, part_hbm.at[wid].at[pl.ds(seg, 1)])

  def load_row(buf, i):
    return [buf[i, pl.ds(k * L, L)] for k in range(NK)]

  def group(buf, vo, carry):
    gvec = idv_v[pl.ds(vo, L)]
    gmin = jnp.min(gvec)
    gmax = jnp.max(gvec)

    def fast(c):
      cur = c[0]
      acc = c[1:]
      bnd = gmin != cur

      @pl.when(bnd)
      def _():
        flush(cur, acc)

      acc = tuple(jnp.where(bnd, zvec, a) for a in acc)

      def add_row(j, ac):
        row = load_row(buf, vo + j)
        return tuple(ac[k] + row[k] for k in range(NK))

      return (gmin,) + lax.fori_loop(0, L, add_row, acc, unroll=2)

    def slow(c):
      def row_fn(i, cc):
        cur = cc[0]
        acc = cc[1:]
        rid = jnp.sum(jnp.where(iota == i - vo, gvec, 0))
        bnd = rid != cur

        @pl.when(bnd)
        def _():
          flush(cur, acc)

        row = load_row(buf, i)
        nacc = tuple(
            jnp.where(bnd, row[k], acc[k] + row[k]) for k in range(NK))
        return (rid,) + nacc

      return lax.fori_loop(vo, vo + L, row_fn, c)

    return lax.cond(gmin == gmax, fast, slow, carry)

  def consume(buf, ngroups, carry):
    def grp(g, cc):
      return group(buf, g * L, cc)

    return lax.fori_loop(0, ngroups, grp, carry)

  def process(s, buf, sem, carry):
    wait(s, buf, sem)
    pltpu.sync_copy(ids_hbm.at[pl.ds(chunk_of(s) * CHUNK, CHUNK)], idv_v)
    return consume(buf, CHUNK // L, carry)

  pltpu.sync_copy(ids_hbm.at[pl.ds(wid * CHUNK, L)], idv_v.at[pl.ds(0, L)])
  first = idv_v[pl.ds(0, L)]
  cur0 = jnp.sum(jnp.where(iota == 0, first, 0))
  carry = (cur0,) + tuple(zvec for _ in range(NK))

  def slot_pair(t, cc):
    s = 2 * t
    fetch(s + 1, rows1_v, sem1)
    cc = process(s, rows0_v, sem0, cc)
    fetch(s + 2, rows0_v, sem0)
    return process(s + 1, rows1_v, sem1, cc)

  carry = lax.fori_loop(0, FULL_SLOTS // 2 - 1, slot_pair, carry)
  fetch(FULL_SLOTS - 1, rows1_v, sem1)
  carry = process(FULL_SLOTS - 2, rows0_v, sem0, carry)
  carry = process(FULL_SLOTS - 1, rows1_v, sem1, carry)

  def slot24(cc):
    def full(c2):
      fetch(FULL_SLOTS, rows0_v, sem0)
      return process(FULL_SLOTS, rows0_v, sem0, c2)

    def short(c2):
      pltpu.sync_copy(x_hbm.at[pl.ds(NFCH * CHUNK, TAILROWS)],
                      rows0_v.at[pl.ds(0, TAILROWS)])
      pltpu.sync_copy(ids_hbm.at[pl.ds(NFCH * CHUNK, TAILROWS)],
                      idv_v.at[pl.ds(0, TAILROWS)])
      return consume(rows0_v, TAILROWS // L, c2)

    return lax.cond(wid == LAST_W, short, full, cc)

  carry = lax.cond(wid <= LAST_W, slot24, lambda c2: c2, carry)

  flush(carry[0], carry[1:])


def _sc_segment_partial(x, ids):
  mesh = plsc.VectorSubcoreMesh(core_axis_name="c", subcore_axis_name="s")
  run = pl.kernel(
      _sc_body,
      out_type=jax.ShapeDtypeStruct((NW, G, D), jnp.float32),
      mesh=mesh,
      scratch_types=[
          pltpu.VMEM((CHUNK, D), jnp.float32),
          pltpu.VMEM((CHUNK, D), jnp.float32),
          pltpu.VMEM((1, D), jnp.float32),
          pltpu.VMEM((ZR, D), jnp.float32),
          pltpu.VMEM((CHUNK,), jnp.int32),
          pltpu.SemaphoreType.DMA,
          pltpu.SemaphoreType.DMA,
      ],
      compiler_params=pltpu.CompilerParams(
          needs_layout_passes=False, use_tc_tiling_on_sc=True),
  )
  return run(x, ids)


def _tc_body(p_ref, a_ref, o_ref):
  pooled = jnp.sum(p_ref[...], axis=0)
  o_ref[...] = jnp.concatenate([pooled, a_ref[...]], axis=1)


def _tc_combine(partial, x_adduct):
  return pl.pallas_call(
      _tc_body,
      out_shape=jax.ShapeDtypeStruct((G, D + DA), jnp.float32),
  )(partial, x_adduct)


@jax.jit
def kernel(x, segment_ids, x_adduct):
  partial = _sc_segment_partial(x, segment_ids)
  return _tc_combine(partial, x_adduct.astype(jnp.float32))


# speedup vs baseline: 1.1142x; 1.1142x over previous
import jax
import jax.numpy as jnp
from jax import lax
from jax.experimental import pallas as pl
from jax.experimental.pallas import tpu as pltpu
from jax.experimental.pallas import tpu_sc as plsc

N = 100000
D = 256
G = 512
DA = 16
L = 16
NK = D // L

NC = 2
NS = 16
NW = NC * NS

CHUNK = 128
NFCH = N // CHUNK
TAILROWS = N - NFCH * CHUNK
FULL_SLOTS = 24
LAST_W = NFCH + 1 - FULL_SLOTS * NW + NW - 1
ZR = 32


def _sc_body(x_hbm, ids_hbm, part_hbm, rows0_v, rows1_v, stage_v, zero_v,
             idv0_v, idv1_v, sem0, sem1, semi0, semi1):
  cid = lax.axis_index("c")
  sid = lax.axis_index("s")
  wid = sid * NC + cid

  iota = lax.iota(jnp.int32, L)
  zvec = jnp.zeros((L,), jnp.float32)

  def chunk_of(s):
    return wid + NW * s

  def fetch(s, buf, sem):
    pltpu.async_copy(x_hbm.at[pl.ds(chunk_of(s) * CHUNK, CHUNK)], buf, sem)

  def wait(s, buf, sem):
    pltpu.make_async_copy(
        x_hbm.at[pl.ds(chunk_of(s) * CHUNK, CHUNK)], buf, sem).wait()

  def fetch_ids(s, ibuf, isem):
    pltpu.async_copy(
        ids_hbm.at[pl.ds(chunk_of(s) * CHUNK, CHUNK)], ibuf, isem)

  def wait_ids(s, ibuf, isem):
    pltpu.make_async_copy(
        ids_hbm.at[pl.ds(chunk_of(s) * CHUNK, CHUNK)], ibuf, isem).wait()

  fetch(0, rows0_v, sem0)
  fetch_ids(0, idv0_v, semi0)

  @pl.loop(0, ZR)
  def _zr(r):
    @pl.loop(0, D, step=L)
    def _zc(j):
      zero_v[r, pl.ds(j, L)] = zvec

  for z in range(G // ZR):
    pltpu.sync_copy(zero_v, part_hbm.at[wid].at[pl.ds(z * ZR, ZR)])

  def flush(seg, acc):
    for k in range(NK):
      stage_v[0, pl.ds(k * L, L)] = acc[k]
    pltpu.sync_copy(stage_v, part_hbm.at[wid].at[pl.ds(seg, 1)])

  def load_row(buf, i):
    return [buf[i, pl.ds(k * L, L)] for k in range(NK)]

  def group(buf, ibuf, vo, carry):
    gvec = ibuf[pl.ds(vo, L)]
    gmin = jnp.min(gvec)
    gmax = jnp.max(gvec)

    def fast(c):
      cur = c[0]
      acc = c[1:]
      bnd = gmin != cur

      @pl.when(bnd)
      def _():
        flush(cur, acc)

      acc = tuple(jnp.where(bnd, zvec, a) for a in acc)

      def add_row(j, ac):
        row = load_row(buf, vo + j)
        return tuple(ac[k] + row[k] for k in range(NK))

      return (gmin,) + lax.fori_loop(0, L, add_row, acc, unroll=2)

    def slow(c):
      def row_fn(i, cc):
        cur = cc[0]
        acc = cc[1:]
        rid = jnp.sum(jnp.where(iota == i - vo, gvec, 0))
        bnd = rid != cur

        @pl.when(bnd)
        def _():
          flush(cur, acc)

        row = load_row(buf, i)
        nacc = tuple(
            jnp.where(bnd, row[k], acc[k] + row[k]) for k in range(NK))
        return (rid,) + nacc

      return lax.fori_loop(vo, vo + L, row_fn, c)

    return lax.cond(gmin == gmax, fast, slow, carry)

  def consume(buf, ibuf, ngroups, carry):
    def grp(g, cc):
      return group(buf, ibuf, g * L, cc)

    return lax.fori_loop(0, ngroups, grp, carry)

  def process(s, buf, sem, ibuf, isem, carry, ids_ready=False):
    wait(s, buf, sem)
    if not ids_ready:
      wait_ids(s, ibuf, isem)
    return consume(buf, ibuf, CHUNK // L, carry)

  carry = (jnp.int32(0),) + tuple(zvec for _ in range(NK))

  def slot_pair(t, cc):
    s = 2 * t
    fetch(s + 1, rows1_v, sem1)
    fetch_ids(s + 1, idv1_v, semi1)
    cc = process(s, rows0_v, sem0, idv0_v, semi0, cc)
    fetch(s + 2, rows0_v, sem0)
    fetch_ids(s + 2, idv0_v, semi0)
    return process(s + 1, rows1_v, sem1, idv1_v, semi1, cc)

  carry = lax.fori_loop(0, FULL_SLOTS // 2 - 1, slot_pair, carry)
  fetch(FULL_SLOTS - 1, rows1_v, sem1)
  fetch_ids(FULL_SLOTS - 1, idv1_v, semi1)
  carry = process(FULL_SLOTS - 2, rows0_v, sem0, idv0_v, semi0, carry)
  carry = process(FULL_SLOTS - 1, rows1_v, sem1, idv1_v, semi1, carry)

  def slot24(cc):
    def full(c2):
      fetch(FULL_SLOTS, rows0_v, sem0)
      fetch_ids(FULL_SLOTS, idv0_v, semi0)
      return process(FULL_SLOTS, rows0_v, sem0, idv0_v, semi0, c2)

    def short(c2):
      pltpu.sync_copy(x_hbm.at[pl.ds(NFCH * CHUNK, TAILROWS)],
                      rows0_v.at[pl.ds(0, TAILROWS)])
      pltpu.sync_copy(ids_hbm.at[pl.ds(NFCH * CHUNK, TAILROWS)],
                      idv0_v.at[pl.ds(0, TAILROWS)])
      return consume(rows0_v, idv0_v, TAILROWS // L, c2)

    return lax.cond(wid == LAST_W, short, full, cc)

  carry = lax.cond(wid <= LAST_W, slot24, lambda c2: c2, carry)

  flush(carry[0], carry[1:])


def _sc_segment_partial(x, ids):
  mesh = plsc.VectorSubcoreMesh(core_axis_name="c", subcore_axis_name="s")
  run = pl.kernel(
      _sc_body,
      out_type=jax.ShapeDtypeStruct((NW, G, D), jnp.float32),
      mesh=mesh,
      scratch_types=[
          pltpu.VMEM((CHUNK, D), jnp.float32),
          pltpu.VMEM((CHUNK, D), jnp.float32),
          pltpu.VMEM((1, D), jnp.float32),
          pltpu.VMEM((ZR, D), jnp.float32),
          pltpu.VMEM((CHUNK,), jnp.int32),
          pltpu.VMEM((CHUNK,), jnp.int32),
          pltpu.SemaphoreType.DMA,
          pltpu.SemaphoreType.DMA,
          pltpu.SemaphoreType.DMA,
          pltpu.SemaphoreType.DMA,
      ],
      compiler_params=pltpu.CompilerParams(
          needs_layout_passes=False, use_tc_tiling_on_sc=True),
  )
  return run(x, ids)


def _tc_body(p_ref, a_ref, o_ref):
  pooled = jnp.sum(p_ref[...], axis=0)
  o_ref[...] = jnp.concatenate([pooled, a_ref[...]], axis=1)


def _tc_combine(partial, x_adduct):
  blk = G // 4
  return pl.pallas_call(
      _tc_body,
      grid=(4,),
      in_specs=[
          pl.BlockSpec((NW, blk, D), lambda i: (0, i, 0)),
          pl.BlockSpec((blk, DA), lambda i: (i, 0)),
      ],
      out_specs=pl.BlockSpec((blk, D + DA), lambda i: (i, 0)),
      out_shape=jax.ShapeDtypeStruct((G, D + DA), jnp.float32),
  )(partial, x_adduct)


@jax.jit
def kernel(x, segment_ids, x_adduct):
  partial = _sc_segment_partial(x, segment_ids)
  return _tc_combine(partial, x_adduct.astype(jnp.float32))
